# baseline (speedup 1.0000x reference)
import functools

import jax
import jax.numpy as jnp
from jax import lax
from jax.experimental import pallas as pl
from jax.experimental.pallas import tpu as pltpu
from jax.experimental.pallas import tpu_sc as plsc

_LANES = 16
_CHUNK = 8
_NBUF = 3


def _connector_sc(x_flat, indices, *, n_rows, n_idx, d):
    num_workers = 32
    rows_per_batch = n_rows // num_workers
    n_chunks = n_idx // _CHUNK
    mesh = plsc.VectorSubcoreMesh(core_axis_name="c", subcore_axis_name="s")

    @functools.partial(
        pl.kernel,
        mesh=mesh,
        out_type=jax.ShapeDtypeStruct((num_workers * n_idx, d), jnp.float32),
        scratch_types=[
            pltpu.VMEM((n_idx,), jnp.int32),
            pltpu.VMEM((_NBUF, _CHUNK, d), jnp.float32),
            pltpu.SemaphoreType.DMA,
            pltpu.SemaphoreType.DMA,
        ],
    )
    def k(x_hbm, idx_hbm, out_hbm, idx_v, rows_v, gsem, ssem):
        wid = lax.axis_index("s") * 2 + lax.axis_index("c")
        pltpu.sync_copy(idx_hbm, idx_v)
        row_base = wid * rows_per_batch
        for i in range(n_idx // _LANES):
            sl = pl.ds(i * _LANES, _LANES)
            idx_v[sl] = idx_v[sl] + row_base

        out_base = wid * n_idx

        def gather(c):
            return pltpu.async_copy(
                x_hbm.at[idx_v.at[pl.ds(c * _CHUNK, _CHUNK)]],
                rows_v.at[c % _NBUF],
                gsem,
            )

        def scatter(c):
            return pltpu.async_copy(
                rows_v.at[c % _NBUF],
                out_hbm.at[pl.ds(out_base + c * _CHUNK, _CHUNK)],
                ssem,
            )

        g = [None] * n_chunks
        s = [None] * n_chunks
        g[0] = gather(0)
        g[1] = gather(1)
        for c in range(n_chunks):
            if c + 2 < n_chunks:
                if c >= 1:
                    s[c - 1].wait()
                g[c + 2] = gather(c + 2)
            g[c].wait()
            s[c] = scatter(c)
        s[n_chunks - 2].wait()
        s[n_chunks - 1].wait()

    return k(x_flat, indices)


def kernel(x, indices):
    b, c, d = x.shape
    (n_idx,) = indices.shape
    x_flat = x.reshape(b * c, d)
    out_flat = _connector_sc(x_flat, indices, n_rows=b * c, n_idx=n_idx, d=d)
    return out_flat.reshape(b, n_idx, d)
